# baseline (speedup 1.0000x reference)
import functools

import jax
import jax.numpy as jnp
from jax import lax
from jax.experimental import pallas as pl
from jax.experimental.pallas import tpu as pltpu
from jax.experimental.pallas import tpu_sc as plsc

NNODE = 10000
NEDGE = 160000
FDIM = 256
FHALF = 128

NCORE = 2
NSUB = 16
LANES = 16
CHUNK = 128
GCHUNK = 256
NCHUNK = 80
NGATH = 40
EPAD = NSUB * NCHUNK * CHUNK
ACC_ROWS = 10240
ROWS_PER_TILE = ACC_ROWS // NSUB

_SC_MESH = plsc.VectorSubcoreMesh(core_axis_name="c", subcore_axis_name="s")


_DEG_SPLIT = NCHUNK // NCORE


@functools.partial(
    pl.kernel,
    out_type=jax.ShapeDtypeStruct((NCORE, ACC_ROWS, FHALF), jnp.float32),
    mesh=_SC_MESH,
    scratch_types=[
        pltpu.VMEM((NCHUNK, CHUNK), jnp.int32),
        pltpu.VMEM((CHUNK, FHALF), jnp.float32),
        pltpu.VMEM_SHARED((ACC_ROWS, FHALF), jnp.float32),
        pltpu.SemaphoreType.DMA,
    ],
)
def _sc_degree(dst_hbm, deg_hbm, dstv, ones_v, acc, sem):
    cid = lax.axis_index("c")
    sid = lax.axis_index("s")
    pltpu.sync_copy(dst_hbm.at[sid], dstv)

    def _fill(val):
        def _f(r, _):
            def _st(c, _):
                ones_v[r, pl.ds(c * LANES, LANES)] = jnp.full((LANES,), val, jnp.float32)
                return 0
            lax.fori_loop(0, FHALF // LANES, _st, 0)
            return 0
        lax.fori_loop(0, CHUNK, _f, 0)

    _fill(0.0)
    base = sid * ROWS_PER_TILE

    def _zero(k, _):
        pltpu.sync_copy(ones_v, acc.at[pl.ds(base + k * CHUNK, CHUNK)])
        return 0

    lax.fori_loop(0, ROWS_PER_TILE // CHUNK, _zero, 0)
    _fill(1.0)
    plsc.subcore_barrier()

    jbase = cid * _DEG_SPLIT

    def _body(j, _):
        pltpu.sync_copy(ones_v, acc.at[dstv.at[jbase + j]], add=True)
        return 0

    lax.fori_loop(0, _DEG_SPLIT, _body, 0)
    plsc.subcore_barrier()

    pltpu.sync_copy(acc.at[pl.ds(base, ROWS_PER_TILE)],
                    deg_hbm.at[cid, pl.ds(base, ROWS_PER_TILE)])


@functools.partial(
    pl.kernel,
    out_type=jax.ShapeDtypeStruct((NCORE, ACC_ROWS, FHALF), jnp.float32),
    mesh=_SC_MESH,
    scratch_types=[
        pltpu.VMEM((GCHUNK,), jnp.int32),
        pltpu.VMEM((16, CHUNK), jnp.int32),
        pltpu.VMEM((GCHUNK, FHALF), jnp.float32),
        pltpu.VMEM_SHARED((ACC_ROWS, FHALF), jnp.float32),
        pltpu.SemaphoreType.DMA,
    ],
)
def _sc_segsum(src_hbm, dst_hbm, tab_hbm, out_hbm, srcv, dstv, rows, acc, sem):
    cid = lax.axis_index("c")
    sid = lax.axis_index("s")

    def _fill0(r, _):
        def _st(c, _):
            rows[r, pl.ds(c * LANES, LANES)] = jnp.zeros((LANES,), jnp.float32)
            return 0
        lax.fori_loop(0, FHALF // LANES, _st, 0)
        return 0

    lax.fori_loop(0, CHUNK, _fill0, 0)
    base = sid * ROWS_PER_TILE

    def _zero(k, _):
        pltpu.sync_copy(rows.at[pl.ds(0, CHUNK)],
                        acc.at[pl.ds(base + k * CHUNK, CHUNK)])
        return 0

    lax.fori_loop(0, ROWS_PER_TILE // CHUNK, _zero, 0)
    plsc.subcore_barrier()

    for q in range(NGATH // 8):
        pltpu.sync_copy(dst_hbm.at[sid, pl.ds(q * 16, 16)], dstv)

        def _body(g, _):
            pltpu.sync_copy(src_hbm.at[cid, sid, q * 8 + g], srcv)
            pltpu.async_copy(tab_hbm.at[srcv], rows, sem).wait()
            pltpu.sync_copy(rows.at[pl.ds(0, CHUNK)],
                            acc.at[dstv.at[2 * g]], add=True)
            pltpu.sync_copy(rows.at[pl.ds(CHUNK, CHUNK)],
                            acc.at[dstv.at[2 * g + 1]], add=True)
            return 0

        lax.fori_loop(0, 8, _body, 0)

    plsc.subcore_barrier()
    pltpu.sync_copy(acc.at[pl.ds(base, ROWS_PER_TILE)],
                    out_hbm.at[cid, pl.ds(base, ROWS_PER_TILE)])


RBLK = 1000
GRID = NNODE // RBLK


def _dis_of(degA, degB):
    deg = degA[0][:, 0:1] + degB[0][:, 0:1] + 1.0
    return lax.rsqrt(jnp.maximum(deg, 1.0))


def _tc1_body(x_ref, w1_ref, degA_ref, degB_ref, h_ref):
    dis = _dis_of(degA_ref[...], degB_ref[...])
    h = jnp.dot(x_ref[...], w1_ref[...], preferred_element_type=jnp.float32)
    hp = h * dis
    h_ref[0] = hp[:, :FHALF]
    h_ref[1] = hp[:, FHALF:]


def _tc2_body(hA_ref, hB_ref, aA_ref, aB_ref, degA_ref, degB_ref,
              w2_ref, b1_ref, g1_ref, be1_ref, o_ref):
    dis = _dis_of(degA_ref[...], degB_ref[...])
    hp = jnp.concatenate([hA_ref[0], hB_ref[0]], axis=1)
    acc = jnp.concatenate([aA_ref[0], aB_ref[0]], axis=1)
    conv = (hp + acc) * dis + b1_ref[...]
    act = jnp.maximum(conv * g1_ref[...] + be1_ref[...], 0.0)
    h2 = jnp.dot(act, w2_ref[...], preferred_element_type=jnp.float32) * dis
    o_ref[0] = h2[:, :FHALF]
    o_ref[1] = h2[:, FHALF:]


def _tc3_body(hA_ref, hB_ref, aA_ref, aB_ref, degA_ref, degB_ref,
              wc_ref, b2_ref, g2_ref, be2_ref, bc_ref, out_ref):
    dis = _dis_of(degA_ref[...], degB_ref[...])
    hp = jnp.concatenate([hA_ref[0], hB_ref[0]], axis=1)
    acc = jnp.concatenate([aA_ref[0], aB_ref[0]], axis=1)
    conv = (hp + acc) * dis + b2_ref[...]
    z = conv * g2_ref[...] + be2_ref[...]
    out_ref[...] = jnp.dot(z, wc_ref[...], preferred_element_type=jnp.float32) + bc_ref[...]


def _row_spec(cols):
    return pl.BlockSpec((RBLK, cols), lambda i: (i, 0))


def _stk_spec(k):
    return pl.BlockSpec((1, RBLK, FHALF), lambda i, k=k: (k, i, 0))


def _half_spec():
    return pl.BlockSpec((2, RBLK, FHALF), lambda i: (0, i, 0))


def _full_spec(shape):
    return pl.BlockSpec(shape, lambda i: (0, 0))


def kernel(x, edge_index, W1, b1, g1, be1, W2, b2, g2, be2, Wc, bc):
    src = edge_index[0]
    dst = edge_index[1]
    pad = EPAD - NEDGE
    src_p = jnp.concatenate([src, jnp.zeros((pad,), jnp.int32)])
    dst_p = jnp.concatenate([dst, jnp.full((pad,), NNODE, jnp.int32)])
    src_r = src_p.reshape(NSUB, NGATH, GCHUNK)
    src2_r = jnp.stack([src_r, src_r + NNODE])
    dst_r = dst_p.reshape(NSUB, NCHUNK, CHUNK)

    deg = _sc_degree(dst_r)

    h12 = pl.pallas_call(
        _tc1_body,
        grid=(GRID,),
        in_specs=[_row_spec(FDIM), _full_spec((FDIM, FDIM)),
                  _stk_spec(0), _stk_spec(1)],
        out_specs=_half_spec(),
        out_shape=jax.ShapeDtypeStruct((2, NNODE, FHALF), jnp.float32),
    )(x, W1, deg, deg)

    acc1 = _sc_segsum(src2_r, dst_r, h12.reshape(2 * NNODE, FHALF))

    b1r = b1.reshape(1, FDIM)
    g1r = g1.reshape(1, FDIM)
    be1r = be1.reshape(1, FDIM)
    h22 = pl.pallas_call(
        _tc2_body,
        grid=(GRID,),
        in_specs=[_stk_spec(0), _stk_spec(1),
                  _stk_spec(0), _stk_spec(1),
                  _stk_spec(0), _stk_spec(1),
                  _full_spec((FDIM, FDIM)),
                  _full_spec((1, FDIM)), _full_spec((1, FDIM)),
                  _full_spec((1, FDIM))],
        out_specs=_half_spec(),
        out_shape=jax.ShapeDtypeStruct((2, NNODE, FHALF), jnp.float32),
    )(h12, h12, acc1, acc1, deg, deg, W2, b1r, g1r, be1r)

    acc2 = _sc_segsum(src2_r, dst_r, h22.reshape(2 * NNODE, FHALF))

    nout = Wc.shape[1]
    wc_p = jnp.zeros((FDIM, FHALF), jnp.float32).at[:, :nout].set(Wc)
    bc_p = jnp.zeros((1, FHALF), jnp.float32).at[0, :nout].set(bc)
    b2r = b2.reshape(1, FDIM)
    g2r = g2.reshape(1, FDIM)
    be2r = be2.reshape(1, FDIM)
    out = pl.pallas_call(
        _tc3_body,
        grid=(GRID,),
        in_specs=[_stk_spec(0), _stk_spec(1),
                  _stk_spec(0), _stk_spec(1),
                  _stk_spec(0), _stk_spec(1),
                  _full_spec((FDIM, FHALF)),
                  _full_spec((1, FDIM)), _full_spec((1, FDIM)),
                  _full_spec((1, FDIM)), _full_spec((1, FHALF))],
        out_specs=_row_spec(FHALF),
        out_shape=jax.ShapeDtypeStruct((NNODE, FHALF), jnp.float32),
    )(h22, h22, acc2, acc2, deg, deg, wc_p, b2r, g2r, be2r, bc_p)

    return out[:, :nout]


# speedup vs baseline: 1.0458x; 1.0458x over previous
import functools

import jax
import jax.numpy as jnp
from jax import lax
from jax.experimental import pallas as pl
from jax.experimental.pallas import tpu as pltpu
from jax.experimental.pallas import tpu_sc as plsc

NNODE = 10000
NEDGE = 160000
FDIM = 256
FHALF = 128

NCORE = 2
NSUB = 16
LANES = 16
CHUNK = 128
GCHUNK = 256
NCHUNK = 80
NGATH = 40
EPAD = NSUB * NCHUNK * CHUNK
ACC_ROWS = 10240
ROWS_PER_TILE = ACC_ROWS // NSUB

_SC_MESH = plsc.VectorSubcoreMesh(core_axis_name="c", subcore_axis_name="s")


_DEG_SPLIT = NCHUNK // NCORE


@functools.partial(
    pl.kernel,
    out_type=jax.ShapeDtypeStruct((NCORE, ACC_ROWS, FHALF), jnp.float32),
    mesh=_SC_MESH,
    scratch_types=[
        pltpu.VMEM((NCHUNK, CHUNK), jnp.int32),
        pltpu.VMEM((CHUNK, FHALF), jnp.float32),
        pltpu.VMEM_SHARED((ACC_ROWS, FHALF), jnp.float32),
        pltpu.SemaphoreType.DMA,
    ],
)
def _sc_degree(dst_hbm, deg_hbm, dstv, ones_v, acc, sem):
    cid = lax.axis_index("c")
    sid = lax.axis_index("s")
    pltpu.sync_copy(dst_hbm.at[sid], dstv)

    def _fill(val):
        def _f(r, _):
            def _st(c, _):
                ones_v[r, pl.ds(c * LANES, LANES)] = jnp.full((LANES,), val, jnp.float32)
                return 0
            lax.fori_loop(0, FHALF // LANES, _st, 0)
            return 0
        lax.fori_loop(0, CHUNK, _f, 0)

    _fill(0.0)
    base = sid * ROWS_PER_TILE

    def _zero(k, _):
        pltpu.sync_copy(ones_v, acc.at[pl.ds(base + k * CHUNK, CHUNK)])
        return 0

    lax.fori_loop(0, ROWS_PER_TILE // CHUNK, _zero, 0)
    _fill(1.0)
    plsc.subcore_barrier()

    jbase = cid * _DEG_SPLIT

    def _body(j, _):
        pltpu.sync_copy(ones_v, acc.at[dstv.at[jbase + j]], add=True)
        return 0

    lax.fori_loop(0, _DEG_SPLIT, _body, 0)
    plsc.subcore_barrier()

    pltpu.sync_copy(acc.at[pl.ds(base, ROWS_PER_TILE)],
                    deg_hbm.at[cid, pl.ds(base, ROWS_PER_TILE)])


@functools.partial(
    pl.kernel,
    out_type=jax.ShapeDtypeStruct((NCORE, ACC_ROWS, FHALF), jnp.float32),
    mesh=_SC_MESH,
    scratch_types=[
        pltpu.VMEM((NGATH * GCHUNK,), jnp.int32),
        pltpu.VMEM((NCHUNK // 2, CHUNK), jnp.int32),
        pltpu.VMEM((GCHUNK, FHALF), jnp.float32),
        pltpu.VMEM_SHARED((ACC_ROWS, FHALF), jnp.float32),
        pltpu.SemaphoreType.DMA,
    ],
)
def _sc_segsum(src_hbm, dst_hbm, tab_hbm, out_hbm, srcv, dstv, rows, acc, sem):
    cid = lax.axis_index("c")
    sid = lax.axis_index("s")
    pltpu.sync_copy(src_hbm.at[cid, sid], srcv)

    def _fill0(r, _):
        def _st(c, _):
            rows[r, pl.ds(c * LANES, LANES)] = jnp.zeros((LANES,), jnp.float32)
            return 0
        lax.fori_loop(0, FHALF // LANES, _st, 0)
        return 0

    lax.fori_loop(0, CHUNK, _fill0, 0)
    base = sid * ROWS_PER_TILE

    def _zero(k, _):
        pltpu.sync_copy(rows.at[pl.ds(0, CHUNK)],
                        acc.at[pl.ds(base + k * CHUNK, CHUNK)])
        return 0

    lax.fori_loop(0, ROWS_PER_TILE // CHUNK, _zero, 0)
    plsc.subcore_barrier()

    for h in range(2):
        pltpu.sync_copy(dst_hbm.at[sid, pl.ds(h * (NCHUNK // 2), NCHUNK // 2)],
                        dstv)

        def _body(g, _):
            j = h * (NGATH // 2) + g
            pltpu.async_copy(tab_hbm.at[srcv.at[pl.ds(j * GCHUNK, GCHUNK)]],
                             rows, sem).wait()
            pltpu.sync_copy(rows.at[pl.ds(0, CHUNK)],
                            acc.at[dstv.at[2 * g]], add=True)
            pltpu.sync_copy(rows.at[pl.ds(CHUNK, CHUNK)],
                            acc.at[dstv.at[2 * g + 1]], add=True)
            return 0

        lax.fori_loop(0, NGATH // 2, _body, 0)

    plsc.subcore_barrier()
    pltpu.sync_copy(acc.at[pl.ds(base, ROWS_PER_TILE)],
                    out_hbm.at[cid, pl.ds(base, ROWS_PER_TILE)])


RBLK = 1000
GRID = NNODE // RBLK


def _dis_of(degA, degB):
    deg = degA[0][:, 0:1] + degB[0][:, 0:1] + 1.0
    return lax.rsqrt(jnp.maximum(deg, 1.0))


def _tc1_body(x_ref, w1_ref, degA_ref, degB_ref, h_ref):
    dis = _dis_of(degA_ref[...], degB_ref[...])
    h = jnp.dot(x_ref[...], w1_ref[...], preferred_element_type=jnp.float32)
    hp = h * dis
    h_ref[0] = hp[:, :FHALF]
    h_ref[1] = hp[:, FHALF:]


def _tc2_body(hA_ref, hB_ref, aA_ref, aB_ref, degA_ref, degB_ref,
              w2_ref, b1_ref, g1_ref, be1_ref, o_ref):
    dis = _dis_of(degA_ref[...], degB_ref[...])
    hp = jnp.concatenate([hA_ref[0], hB_ref[0]], axis=1)
    acc = jnp.concatenate([aA_ref[0], aB_ref[0]], axis=1)
    conv = (hp + acc) * dis + b1_ref[...]
    act = jnp.maximum(conv * g1_ref[...] + be1_ref[...], 0.0)
    h2 = jnp.dot(act, w2_ref[...], preferred_element_type=jnp.float32) * dis
    o_ref[0] = h2[:, :FHALF]
    o_ref[1] = h2[:, FHALF:]


def _tc3_body(hA_ref, hB_ref, aA_ref, aB_ref, degA_ref, degB_ref,
              wc_ref, b2_ref, g2_ref, be2_ref, bc_ref, out_ref):
    dis = _dis_of(degA_ref[...], degB_ref[...])
    hp = jnp.concatenate([hA_ref[0], hB_ref[0]], axis=1)
    acc = jnp.concatenate([aA_ref[0], aB_ref[0]], axis=1)
    conv = (hp + acc) * dis + b2_ref[...]
    z = conv * g2_ref[...] + be2_ref[...]
    out_ref[...] = jnp.dot(z, wc_ref[...], preferred_element_type=jnp.float32) + bc_ref[...]


def _row_spec(cols):
    return pl.BlockSpec((RBLK, cols), lambda i: (i, 0))


def _stk_spec(k):
    return pl.BlockSpec((1, RBLK, FHALF), lambda i, k=k: (k, i, 0))


def _half_spec():
    return pl.BlockSpec((2, RBLK, FHALF), lambda i: (0, i, 0))


def _full_spec(shape):
    return pl.BlockSpec(shape, lambda i: (0, 0))


def kernel(x, edge_index, W1, b1, g1, be1, W2, b2, g2, be2, Wc, bc):
    src = edge_index[0]
    dst = edge_index[1]
    pad = EPAD - NEDGE
    src_p = jnp.concatenate([src, jnp.zeros((pad,), jnp.int32)])
    dst_p = jnp.concatenate([dst, jnp.full((pad,), NNODE, jnp.int32)])
    src_r = src_p.reshape(NSUB, NGATH * GCHUNK)
    src2_r = jnp.stack([src_r, src_r + NNODE])
    dst_r = dst_p.reshape(NSUB, NCHUNK, CHUNK)

    deg = _sc_degree(dst_r)

    h12 = pl.pallas_call(
        _tc1_body,
        grid=(GRID,),
        in_specs=[_row_spec(FDIM), _full_spec((FDIM, FDIM)),
                  _stk_spec(0), _stk_spec(1)],
        out_specs=_half_spec(),
        out_shape=jax.ShapeDtypeStruct((2, NNODE, FHALF), jnp.float32),
    )(x, W1, deg, deg)

    acc1 = _sc_segsum(src2_r, dst_r, h12.reshape(2 * NNODE, FHALF))

    b1r = b1.reshape(1, FDIM)
    g1r = g1.reshape(1, FDIM)
    be1r = be1.reshape(1, FDIM)
    h22 = pl.pallas_call(
        _tc2_body,
        grid=(GRID,),
        in_specs=[_stk_spec(0), _stk_spec(1),
                  _stk_spec(0), _stk_spec(1),
                  _stk_spec(0), _stk_spec(1),
                  _full_spec((FDIM, FDIM)),
                  _full_spec((1, FDIM)), _full_spec((1, FDIM)),
                  _full_spec((1, FDIM))],
        out_specs=_half_spec(),
        out_shape=jax.ShapeDtypeStruct((2, NNODE, FHALF), jnp.float32),
    )(h12, h12, acc1, acc1, deg, deg, W2, b1r, g1r, be1r)

    acc2 = _sc_segsum(src2_r, dst_r, h22.reshape(2 * NNODE, FHALF))

    nout = Wc.shape[1]
    wc_p = jnp.zeros((FDIM, FHALF), jnp.float32).at[:, :nout].set(Wc)
    bc_p = jnp.zeros((1, FHALF), jnp.float32).at[0, :nout].set(bc)
    b2r = b2.reshape(1, FDIM)
    g2r = g2.reshape(1, FDIM)
    be2r = be2.reshape(1, FDIM)
    out = pl.pallas_call(
        _tc3_body,
        grid=(GRID,),
        in_specs=[_stk_spec(0), _stk_spec(1),
                  _stk_spec(0), _stk_spec(1),
                  _stk_spec(0), _stk_spec(1),
                  _full_spec((FDIM, FHALF)),
                  _full_spec((1, FDIM)), _full_spec((1, FDIM)),
                  _full_spec((1, FDIM)), _full_spec((1, FHALF))],
        out_specs=_row_spec(FHALF),
        out_shape=jax.ShapeDtypeStruct((NNODE, FHALF), jnp.float32),
    )(h22, h22, acc2, acc2, deg, deg, wc_p, b2r, g2r, be2r, bc_p)

    return out[:, :nout]
